# baseline (speedup 1.0000x reference)
import functools

import jax
import jax.numpy as jnp
from jax import lax
from jax.experimental import pallas as pl
from jax.experimental.pallas import tpu as pltpu
from jax.experimental.pallas import tpu_sc as plsc

N = 10000
E = 320000
D = 128

NC = 2
NS = 16
NW = NC * NS

CH = 128
EPAD = -(-E // (NW * CH)) * (NW * CH)
NCHUNK = EPAD // CH
NFULL = NCHUNK // NW
PKW = 3 * CH
NB = 3

NRC = N // CH
RTAIL = N - NRC * CH

_mesh = plsc.VectorSubcoreMesh(core_axis_name="c", subcore_axis_name="s")


@functools.partial(
    pl.kernel,
    mesh=_mesh,
    out_type=jax.ShapeDtypeStruct((NC, N, D), jnp.float32),
    scratch_types=[
        pltpu.VMEM((CH, D), jnp.float32),
        pltpu.VMEM((CH, D), jnp.float32),
        pltpu.VMEM((CH, D), jnp.float32),
        pltpu.VMEM((1, PKW), jnp.int32),
        pltpu.VMEM((1, PKW), jnp.int32),
        pltpu.VMEM((1, PKW), jnp.int32),
        pltpu.VMEM((CH,), jnp.int32),
        pltpu.VMEM((CH,), jnp.int32),
        pltpu.VMEM((CH,), jnp.int32),
        pltpu.VMEM_SHARED((N, D), jnp.float32),
        pltpu.SemaphoreType.DMA,
        pltpu.SemaphoreType.DMA,
        pltpu.SemaphoreType.DMA,
        pltpu.SemaphoreType.DMA,
        pltpu.SemaphoreType.DMA,
        pltpu.SemaphoreType.DMA,
        pltpu.SemaphoreType.DMA,
        pltpu.SemaphoreType.DMA,
        pltpu.SemaphoreType.DMA,
    ],
)
def _spmm(x_hbm, epk_hbm, out_hbm,
          r0, r1, r2, p0, p1, p2, i0, i1, i2, agg,
          sg0, sg1, sg2, ss0, ss1, ss2, si0, si1, si2):
    c = lax.axis_index("c")
    s = lax.axis_index("s")
    wid = c * NS + s
    cbase = wid * NFULL

    rows = (r0, r1, r2)
    pk = (p0, p1, p2)
    ridx = (i0, i1, i2)
    sg = (sg0, sg1, sg2)
    ss = (ss0, ss1, ss2)
    si = (si0, si1, si2)

    def _zero(i, _):
        for j in range(8):
            r0[i, pl.ds(j * 16, 16)] = jnp.zeros((16,), jnp.float32)
        return 0
    lax.fori_loop(0, CH, _zero, 0)

    for k in range(NRC // NS + 1):
        q = s + NS * k
        @pl.when(q < NRC)
        def _():
            pltpu.sync_copy(r0, agg.at[pl.ds(q * CH, CH)])
    @pl.when(s == 0)
    def _():
        pltpu.sync_copy(r0.at[pl.ds(0, RTAIL)], agg.at[pl.ds(NRC * CH, RTAIL)])
    plsc.subcore_barrier()

    def _stage(cc, b):
        pltpu.async_copy(epk_hbm.at[cbase + cc], pk[b], si[b])

    def _wait_stage(b):
        pltpu.make_async_copy(epk_hbm.at[0], pk[b], si[b]).wait()

    def _scale(b, sz):
        def body(g, _):
            w16i = pk[b][0, pl.ds(2 * CH + g * 16, 16)]
            w16 = lax.bitcast_convert_type(w16i, jnp.float32)
            for j in range(16):
                wj = w16[j]
                e = g * 16 + j
                for k in range(8):
                    rows[b][e, pl.ds(k * 16, 16)] = (
                        rows[b][e, pl.ds(k * 16, 16)] * wj)
            return 0
        lax.fori_loop(0, sz // 16, body, 0)

    _stage(0, 0)
    _stage(1, 1)
    _wait_stage(0)
    pltpu.async_copy(x_hbm.at[p0.at[0, pl.ds(0, CH)]], r0, sg0)

    def _visit(v, b):
        bn = (b + 1) % NB
        bs = (b + 2) % NB
        @pl.when((v >= 1) & (v <= NFULL))
        def _():
            pltpu.make_async_copy(rows[bs], agg.at[ridx[bs]], ss[bs]).wait()
        @pl.when(v + 2 < NFULL)
        def _():
            _stage(v + 2, bs)
        @pl.when(v + 1 < NFULL)
        def _():
            _wait_stage(bn)
            pltpu.async_copy(x_hbm.at[pk[bn].at[0, pl.ds(0, CH)]],
                             rows[bn], sg[bn])
        @pl.when(v < NFULL)
        def _():
            pltpu.make_async_copy(x_hbm.at[pk[b].at[0, pl.ds(0, CH)]],
                                  rows[b], sg[b]).wait()
            for k in range(CH // 16):
                ridx[b][pl.ds(k * 16, 16)] = pk[b][0, pl.ds(CH + k * 16, 16)]
            _scale(b, CH)
            pltpu.async_copy(rows[b], agg.at[ridx[b]], ss[b], add=True)

    def _triple(g, _):
        for j in range(NB):
            _visit(g * NB + j, j)
        return 0
    lax.fori_loop(0, (NFULL + NB - 1) // NB + 1, _triple, 0)

    plsc.subcore_barrier()

    for k in range(NRC // NS + 1):
        q = s + NS * k
        @pl.when(q < NRC)
        def _():
            pltpu.sync_copy(agg.at[pl.ds(q * CH, CH)], r0)
            pltpu.sync_copy(r0, out_hbm.at[c, pl.ds(q * CH, CH)])
    @pl.when(s == 0)
    def _():
        pltpu.sync_copy(agg.at[pl.ds(NRC * CH, RTAIL)], r1.at[pl.ds(0, RTAIL)])
        pltpu.sync_copy(r1.at[pl.ds(0, RTAIL)],
                        out_hbm.at[c, pl.ds(NRC * CH, RTAIL)])


def _tc_body(p_ref, wt_ref, b_ref, o_ref):
    ssum = p_ref[0] + p_ref[1]
    o_ref[...] = jnp.dot(ssum, wt_ref[...],
                         preferred_element_type=jnp.float32,
                         precision=lax.Precision.HIGHEST) + b_ref[...]


_linear = pl.pallas_call(
    _tc_body,
    grid=(10,),
    in_specs=[
        pl.BlockSpec((NC, N // 10, D), lambda i: (0, i, 0)),
        pl.BlockSpec((D, D), lambda i: (0, 0)),
        pl.BlockSpec((1, D), lambda i: (0, 0)),
    ],
    out_specs=pl.BlockSpec((N // 10, D), lambda i: (i, 0)),
    out_shape=jax.ShapeDtypeStruct((N, D), jnp.float32),
)


def kernel(x, edge_index, edge_weight, W, b):
    row = edge_index[0].astype(jnp.int32)
    col = edge_index[1].astype(jnp.int32)
    npad = EPAD - E
    col_p = jnp.concatenate([col, jnp.zeros((npad,), jnp.int32)])
    row_p = jnp.concatenate([row, jnp.zeros((npad,), jnp.int32)])
    w_p = jnp.concatenate([jax.lax.bitcast_convert_type(edge_weight,
                                                        jnp.int32),
                           jnp.zeros((npad,), jnp.int32)])
    epk = jnp.concatenate([col_p.reshape(NCHUNK, CH),
                           row_p.reshape(NCHUNK, CH),
                           w_p.reshape(NCHUNK, CH)],
                          axis=1).reshape(NCHUNK, 1, PKW)
    partials = _spmm(x, epk)
    return _linear(partials, W.T, b[None, :])


# speedup vs baseline: 1.7792x; 1.7792x over previous
import functools

import jax
import jax.numpy as jnp
from jax import lax
from jax.experimental import pallas as pl
from jax.experimental.pallas import tpu as pltpu
from jax.experimental.pallas import tpu_sc as plsc

N = 10000
E = 320000
D = 128

NC = 2
NS = 16
NW = NC * NS

EPT = E // NW
CH = 128
NFULL = EPT // CH
TAIL = EPT - NFULL * CH
NB = 3

_mesh = plsc.VectorSubcoreMesh(core_axis_name="c", subcore_axis_name="s")


@functools.partial(
    pl.kernel,
    mesh=_mesh,
    out_type=jax.ShapeDtypeStruct((NC, N, D), jnp.float32),
    scratch_types=[
        pltpu.VMEM((CH, D), jnp.float32),
        pltpu.VMEM((CH, D), jnp.float32),
        pltpu.VMEM((CH, D), jnp.float32),
        pltpu.VMEM((CH,), jnp.int32),
        pltpu.VMEM((CH,), jnp.int32),
        pltpu.VMEM((CH,), jnp.int32),
        pltpu.VMEM((CH,), jnp.int32),
        pltpu.VMEM((CH,), jnp.int32),
        pltpu.VMEM((CH,), jnp.int32),
        pltpu.VMEM((CH,), jnp.float32),
        pltpu.VMEM((CH,), jnp.float32),
        pltpu.VMEM((CH,), jnp.float32),
        pltpu.VMEM((TAIL,), jnp.int32),
        pltpu.VMEM((TAIL,), jnp.int32),
        pltpu.VMEM_SHARED((N, D), jnp.float32),
        pltpu.SemaphoreType.DMA,
        pltpu.SemaphoreType.DMA,
        pltpu.SemaphoreType.DMA,
        pltpu.SemaphoreType.DMA,
        pltpu.SemaphoreType.DMA,
        pltpu.SemaphoreType.DMA,
        pltpu.SemaphoreType.DMA,
        pltpu.SemaphoreType.DMA,
        pltpu.SemaphoreType.DMA,
        pltpu.SemaphoreType.DMA,
    ],
)
def _spmm(x_hbm, row_hbm, col_hbm, w_hbm, out_hbm,
          r0, r1, r2, c0, c1, c2, i0, i1, i2, w0, w1, w2,
          colt, rowt, agg,
          sg0, sg1, sg2, ss0, ss1, ss2, si0, si1, si2, sem):
    c = lax.axis_index("c")
    s = lax.axis_index("s")
    wid = c * NS + s
    ebase = pl.multiple_of(wid * EPT, 8)

    rows = (r0, r1, r2)
    colv = (c0, c1, c2)
    ridx = (i0, i1, i2)
    wv = (w0, w1, w2)
    sg = (sg0, sg1, sg2)
    ss = (ss0, ss1, ss2)
    si = (si0, si1, si2)

    def _zero(i, _):
        for j in range(8):
            r0[i, pl.ds(j * 16, 16)] = jnp.zeros((16,), jnp.float32)
        return 0
    lax.fori_loop(0, CH, _zero, 0)

    NRC = N // CH
    RTAIL = N - NRC * CH
    for k in range(NRC // NS + 1):
        q = s + NS * k
        @pl.when(q < NRC)
        def _():
            pltpu.sync_copy(r0, agg.at[pl.ds(q * CH, CH)])
    @pl.when(s == 0)
    def _():
        pltpu.sync_copy(r0.at[pl.ds(0, RTAIL)], agg.at[pl.ds(NRC * CH, RTAIL)])
    plsc.subcore_barrier()

    def _stage(cc, b):
        off = pl.multiple_of(ebase + cc * CH, 8)
        pltpu.async_copy(col_hbm.at[pl.ds(off, CH)], colv[b], si[b])
        pltpu.async_copy(row_hbm.at[pl.ds(off, CH)], ridx[b], si[b])
        pltpu.async_copy(w_hbm.at[pl.ds(off, CH)], wv[b], si[b])

    def _wait_stage(b):
        pltpu.make_async_copy(col_hbm.at[pl.ds(0, CH)], colv[b], si[b]).wait()
        pltpu.make_async_copy(row_hbm.at[pl.ds(0, CH)], ridx[b], si[b]).wait()
        pltpu.make_async_copy(w_hbm.at[pl.ds(0, CH)], wv[b], si[b]).wait()

    def _wait_scatter(b):
        pltpu.make_async_copy(rows[b], agg.at[ridx[b]], ss[b]).wait()

    def _scale(b, sz):
        def body(g, _):
            w16 = wv[b][pl.ds(g * 16, 16)]
            for j in range(16):
                wj = w16[j]
                e = g * 16 + j
                for k in range(8):
                    rows[b][e, pl.ds(k * 16, 16)] = (
                        rows[b][e, pl.ds(k * 16, 16)] * wj)
            return 0
        lax.fori_loop(0, sz // 16, body, 0)

    _stage(0, 0)
    _stage(1, 1)
    _wait_stage(0)
    pltpu.async_copy(x_hbm.at[colv[0]], rows[0], sg[0])

    def _visit(v, b):
        bn = (b + 1) % NB
        bs = (b + 2) % NB
        @pl.when(v + 1 < NFULL)
        def _():
            _wait_stage(bn)
            pltpu.async_copy(x_hbm.at[colv[bn]], rows[bn], sg[bn])
        pltpu.make_async_copy(x_hbm.at[colv[b]], rows[b], sg[b]).wait()
        _scale(b, CH)
        @pl.when(v >= 1)
        def _():
            _wait_scatter(bs)
        @pl.when(v + 2 < NFULL)
        def _():
            _stage(v + 2, bs)
        pltpu.async_copy(rows[b], agg.at[ridx[b]], ss[b], add=True)

    def _triple(g, _):
        for j in range(NB):
            _visit(g * NB + j, j)
        return 0
    lax.fori_loop(0, NFULL // NB, _triple, 0)

    _wait_scatter((NFULL - 1) % NB)

    toff = pl.multiple_of(ebase + NFULL * CH, 8)
    pltpu.sync_copy(col_hbm.at[pl.ds(toff, TAIL)], colt)
    pltpu.sync_copy(row_hbm.at[pl.ds(toff, TAIL)], rowt)
    pltpu.sync_copy(w_hbm.at[pl.ds(toff, TAIL)], wv[0].at[pl.ds(0, TAIL)])
    pltpu.async_copy(x_hbm.at[colt], rows[0].at[pl.ds(0, TAIL)], sem).wait()
    w16 = wv[0][pl.ds(0, TAIL)]
    for j in range(TAIL):
        wj = w16[j]
        for k in range(8):
            rows[0][j, pl.ds(k * 16, 16)] = rows[0][j, pl.ds(k * 16, 16)] * wj
    pltpu.sync_copy(rows[0].at[pl.ds(0, TAIL)], agg.at[rowt], add=True)

    plsc.subcore_barrier()

    for k in range(NRC // NS + 1):
        q = s + NS * k
        @pl.when(q < NRC)
        def _():
            pltpu.sync_copy(agg.at[pl.ds(q * CH, CH)], r0)
            pltpu.sync_copy(r0, out_hbm.at[c, pl.ds(q * CH, CH)])
    @pl.when(s == 0)
    def _():
        pltpu.sync_copy(agg.at[pl.ds(NRC * CH, RTAIL)], r1.at[pl.ds(0, RTAIL)])
        pltpu.sync_copy(r1.at[pl.ds(0, RTAIL)], out_hbm.at[c, pl.ds(NRC * CH, RTAIL)])


def _tc_body(p_ref, wt_ref, b_ref, o_ref):
    ssum = p_ref[0] + p_ref[1]
    o_ref[...] = jnp.dot(ssum, wt_ref[...],
                         preferred_element_type=jnp.float32) + b_ref[...]


_linear = pl.pallas_call(
    _tc_body,
    grid=(10,),
    in_specs=[
        pl.BlockSpec((NC, N // 10, D), lambda i: (0, i, 0)),
        pl.BlockSpec((D, D), lambda i: (0, 0)),
        pl.BlockSpec((1, D), lambda i: (0, 0)),
    ],
    out_specs=pl.BlockSpec((N // 10, D), lambda i: (i, 0)),
    out_shape=jax.ShapeDtypeStruct((N, D), jnp.float32),
)


def kernel(x, edge_index, edge_weight, W, b):
    row = edge_index[0].astype(jnp.int32)
    col = edge_index[1].astype(jnp.int32)
    partials = _spmm(x, row, col, edge_weight)
    return _linear(partials, W.T, b[None, :])


# speedup vs baseline: 1.7829x; 1.0021x over previous
import functools

import jax
import jax.numpy as jnp
from jax import lax
from jax.experimental import pallas as pl
from jax.experimental.pallas import tpu as pltpu
from jax.experimental.pallas import tpu_sc as plsc

N = 10000
E = 320000
D = 128

NC = 2
NS = 16
NW = NC * NS

EPT = E // NW
CH = 96
NFULL = EPT // CH
TAIL = EPT - NFULL * CH
NRC = N // CH
RTAIL = N - NRC * CH

_mesh = plsc.VectorSubcoreMesh(core_axis_name="c", subcore_axis_name="s")


@functools.partial(
    pl.kernel,
    mesh=_mesh,
    out_type=jax.ShapeDtypeStruct((NC, N, D), jnp.float32),
    scratch_types=[
        pltpu.VMEM((CH, D), jnp.float32),
        pltpu.VMEM((CH, D), jnp.float32),
        pltpu.VMEM((CH, D), jnp.float32),
        pltpu.VMEM((CH, D), jnp.float32),
        pltpu.VMEM((CH,), jnp.int32),
        pltpu.VMEM((CH,), jnp.int32),
        pltpu.VMEM((CH,), jnp.int32),
        pltpu.VMEM((CH,), jnp.int32),
        pltpu.VMEM((CH,), jnp.float32),
        pltpu.VMEM((CH,), jnp.float32),
        pltpu.VMEM((CH,), jnp.float32),
        pltpu.VMEM((CH,), jnp.float32),
        pltpu.VMEM((CH,), jnp.int32),
        pltpu.VMEM((CH,), jnp.int32),
        pltpu.VMEM((CH,), jnp.int32),
        pltpu.VMEM((CH,), jnp.int32),
        pltpu.VMEM((TAIL,), jnp.int32),
        pltpu.VMEM((TAIL,), jnp.int32),
        pltpu.VMEM_SHARED((N, D), jnp.float32),
        pltpu.SemaphoreType.DMA,
        pltpu.SemaphoreType.DMA,
        pltpu.SemaphoreType.DMA,
        pltpu.SemaphoreType.DMA,
        pltpu.SemaphoreType.DMA,
        pltpu.SemaphoreType.DMA,
        pltpu.SemaphoreType.DMA,
        pltpu.SemaphoreType.DMA,
        pltpu.SemaphoreType.DMA,
        pltpu.SemaphoreType.DMA,
        pltpu.SemaphoreType.DMA,
        pltpu.SemaphoreType.DMA,
        pltpu.SemaphoreType.DMA,
    ],
)
def _spmm(x_hbm, row_hbm, col_hbm, w_hbm, out_hbm,
          r0, r1, r2, r3, c0, c1, c2, c3, w0, w1, w2, w3,
          i0, i1, i2, i3,
          colt, rowt, agg,
          sg0, sg1, sg2, sg3, ss0, ss1, ss2, ss3,
          si0, si1, si2, si3, sem):
    c = lax.axis_index("c")
    s = lax.axis_index("s")
    wid = c * NS + s
    ebase = pl.multiple_of(wid * EPT, 8)

    rows = (r0, r1, r2, r3)
    colv = (c0, c1, c2, c3)
    wv = (w0, w1, w2, w3)
    ridx = (i0, i1, i2, i3)
    sg = (sg0, sg1, sg2, sg3)
    ss = (ss0, ss1, ss2, ss3)
    si = (si0, si1, si2, si3)

    def _zero(i, _):
        for j in range(8):
            r0[i, pl.ds(j * 16, 16)] = jnp.zeros((16,), jnp.float32)
        return 0
    lax.fori_loop(0, CH, _zero, 0)

    for k in range(NRC // NS + 1):
        q = s + NS * k
        @pl.when(q < NRC)
        def _():
            pltpu.sync_copy(r0, agg.at[pl.ds(q * CH, CH)])
    @pl.when(s == 0)
    def _():
        pltpu.sync_copy(r0.at[pl.ds(0, RTAIL)], agg.at[pl.ds(NRC * CH, RTAIL)])
    plsc.subcore_barrier()

    def _stage(cc, b):
        off = pl.multiple_of(ebase + cc * CH, 8)
        pltpu.async_copy(col_hbm.at[pl.ds(off, CH)], colv[b], si[b])
        pltpu.async_copy(row_hbm.at[pl.ds(off, CH)], ridx[b], si[b])
        pltpu.async_copy(w_hbm.at[pl.ds(off, CH)], wv[b], si[b])

    def _wait_stage(b):
        pltpu.make_async_copy(col_hbm.at[pl.ds(0, CH)], colv[b], si[b]).wait()
        pltpu.make_async_copy(row_hbm.at[pl.ds(0, CH)], ridx[b], si[b]).wait()
        pltpu.make_async_copy(w_hbm.at[pl.ds(0, CH)], wv[b], si[b]).wait()

    def _wait_scatter(b):
        pltpu.make_async_copy(rows[b], agg.at[ridx[b]], ss[b]).wait()

    def _scale(b, sz):
        def body(g, _):
            w16 = wv[b][pl.ds(g * 16, 16)]
            for j in range(16):
                wj = w16[j]
                e = g * 16 + j
                for k in range(8):
                    rows[b][e, pl.ds(k * 16, 16)] = (
                        rows[b][e, pl.ds(k * 16, 16)] * wj)
            return 0
        lax.fori_loop(0, sz // 16, body, 0)

    _stage(0, 0)
    _stage(1, 1)
    _stage(2, 2)
    _wait_stage(0)
    pltpu.async_copy(x_hbm.at[colv[0]], rows[0], sg[0])
    _wait_stage(1)
    pltpu.async_copy(x_hbm.at[colv[1]], rows[1], sg[1])

    def _visit(v, j):
        b = j % 4
        g4 = (j + 2) % 4
        n4 = (j + 3) % 4
        @pl.when(v + 2 < NFULL)
        def _():
            _wait_stage(g4)
            pltpu.async_copy(x_hbm.at[colv[g4]], rows[g4], sg[g4])
        pltpu.make_async_copy(x_hbm.at[colv[b]], rows[b], sg[b]).wait()
        _scale(b, CH)
        @pl.when(v >= 1)
        def _():
            _wait_scatter(n4)
        @pl.when(v + 3 < NFULL)
        def _():
            _stage(v + 3, n4)
        pltpu.async_copy(rows[b], agg.at[ridx[b]], ss[b], add=True)

    def _quad(g, _):
        for j in range(4):
            _visit(g * 4 + j, j)
        return 0
    lax.fori_loop(0, NFULL // 4, _quad, 0)

    _wait_scatter((NFULL - 1) % 4)

    toff = pl.multiple_of(ebase + NFULL * CH, 8)
    pltpu.sync_copy(col_hbm.at[pl.ds(toff, TAIL)], colt)
    pltpu.sync_copy(row_hbm.at[pl.ds(toff, TAIL)], rowt)
    pltpu.sync_copy(w_hbm.at[pl.ds(toff, TAIL)], wv[0].at[pl.ds(0, TAIL)])
    pltpu.async_copy(x_hbm.at[colt], rows[0].at[pl.ds(0, TAIL)], sem).wait()
    w16 = wv[0][pl.ds(0, TAIL)]
    for j in range(TAIL):
        wj = w16[j]
        for k in range(8):
            rows[0][j, pl.ds(k * 16, 16)] = rows[0][j, pl.ds(k * 16, 16)] * wj
    pltpu.sync_copy(rows[0].at[pl.ds(0, TAIL)], agg.at[rowt], add=True)

    plsc.subcore_barrier()

    for k in range(NRC // NS + 1):
        q = s + NS * k
        @pl.when(q < NRC)
        def _():
            pltpu.sync_copy(agg.at[pl.ds(q * CH, CH)], r0)
            pltpu.sync_copy(r0, out_hbm.at[c, pl.ds(q * CH, CH)])
    @pl.when(s == 0)
    def _():
        pltpu.sync_copy(agg.at[pl.ds(NRC * CH, RTAIL)], r1.at[pl.ds(0, RTAIL)])
        pltpu.sync_copy(r1.at[pl.ds(0, RTAIL)],
                        out_hbm.at[c, pl.ds(NRC * CH, RTAIL)])


def _tc_body(p_ref, wt_ref, b_ref, o_ref):
    ssum = p_ref[0] + p_ref[1]
    o_ref[...] = jnp.dot(ssum, wt_ref[...],
                         preferred_element_type=jnp.float32) + b_ref[...]


_linear = pl.pallas_call(
    _tc_body,
    grid=(10,),
    in_specs=[
        pl.BlockSpec((NC, N // 10, D), lambda i: (0, i, 0)),
        pl.BlockSpec((D, D), lambda i: (0, 0)),
        pl.BlockSpec((1, D), lambda i: (0, 0)),
    ],
    out_specs=pl.BlockSpec((N // 10, D), lambda i: (i, 0)),
    out_shape=jax.ShapeDtypeStruct((N, D), jnp.float32),
)


def kernel(x, edge_index, edge_weight, W, b):
    row = edge_index[0].astype(jnp.int32)
    col = edge_index[1].astype(jnp.int32)
    partials = _spmm(x, row, col, edge_weight)
    return _linear(partials, W.T, b[None, :])
